# baseline (speedup 1.0000x reference)
import functools

import jax
import jax.numpy as jnp
from jax import lax
from jax.experimental import pallas as pl
from jax.experimental.pallas import tpu as pltpu
from jax.experimental.pallas import tpu_sc as plsc

LENGTH = 8192
N_DIM = 1024

_info = plsc.get_sparse_core_info()
_NC, _NS = _info.num_cores, _info.num_subcores
_NW = _NC * _NS
_ROWS_PER_W = LENGTH // _NW

_CHUNK = 16
_NBUF = 6
_NCHUNK = _ROWS_PER_W // _CHUNK

_mesh = plsc.VectorSubcoreMesh(core_axis_name="c", subcore_axis_name="s")


@functools.partial(
    pl.kernel,
    out_type=jax.ShapeDtypeStruct((1, LENGTH, N_DIM), jnp.float32),
    mesh=_mesh,
    scratch_types=(
        [pltpu.VMEM((_CHUNK, N_DIM), jnp.float32) for _ in range(_NBUF)]
        + [pltpu.SemaphoreType.DMA for _ in range(2 * _NBUF)]
    ),
)
def _copy_rows(table_hbm, out_hbm, *scratch):
    bufs = scratch[:_NBUF]
    in_sems = scratch[_NBUF:2 * _NBUF]
    out_sems = scratch[2 * _NBUF:]

    wid = lax.axis_index("s") * _NC + lax.axis_index("c")
    base = wid * _ROWS_PER_W

    def start_in(c):
        return pltpu.async_copy(
            table_hbm.at[pl.ds(base + c * _CHUNK, _CHUNK), :],
            bufs[c % _NBUF],
            in_sems[c % _NBUF],
        )

    def start_out(c):
        return pltpu.async_copy(
            bufs[c % _NBUF],
            out_hbm.at[0, pl.ds(base + c * _CHUNK, _CHUNK), :],
            out_sems[c % _NBUF],
        )

    in_cp = {}
    out_cp = {}
    for b in range(_NBUF - 1):
        in_cp[b] = start_in(b)
    for c in range(_NCHUNK):
        in_cp[c].wait()
        out_cp[c] = start_out(c)
        n = c + _NBUF - 1
        if n < _NCHUNK:
            if c >= 1:
                out_cp[c - 1].wait()
            in_cp[n] = start_in(n)
    for c in range(_NCHUNK - _NBUF, _NCHUNK):
        if c >= 0:
            out_cp[c].wait()


def kernel(x, embed_weight):
    return _copy_rows(embed_weight)


# speedup vs baseline: 1.0067x; 1.0067x over previous
import functools

import jax
import jax.numpy as jnp
from jax import lax
from jax.experimental import pallas as pl
from jax.experimental.pallas import tpu as pltpu
from jax.experimental.pallas import tpu_sc as plsc

LENGTH = 8192
N_DIM = 1024

_info = plsc.get_sparse_core_info()
_NC, _NS = _info.num_cores, _info.num_subcores
_NW = _NC * _NS
_ROWS_PER_W = LENGTH // _NW

_CHUNK = 16
_NBUF = 7
_NCHUNK = _ROWS_PER_W // _CHUNK

_mesh = plsc.VectorSubcoreMesh(core_axis_name="c", subcore_axis_name="s")


@functools.partial(
    pl.kernel,
    out_type=jax.ShapeDtypeStruct((1, LENGTH, N_DIM), jnp.float32),
    mesh=_mesh,
    scratch_types=(
        [pltpu.VMEM((_CHUNK, N_DIM), jnp.float32) for _ in range(_NBUF)]
        + [pltpu.SemaphoreType.DMA for _ in range(2 * _NBUF)]
    ),
)
def _copy_rows(table_hbm, out_hbm, *scratch):
    bufs = scratch[:_NBUF]
    in_sems = scratch[_NBUF:2 * _NBUF]
    out_sems = scratch[2 * _NBUF:]

    wid = lax.axis_index("s") * _NC + lax.axis_index("c")
    base = wid * _ROWS_PER_W

    def start_in(c):
        return pltpu.async_copy(
            table_hbm.at[pl.ds(base + c * _CHUNK, _CHUNK), :],
            bufs[c % _NBUF],
            in_sems[c % _NBUF],
        )

    def start_out(c):
        return pltpu.async_copy(
            bufs[c % _NBUF],
            out_hbm.at[0, pl.ds(base + c * _CHUNK, _CHUNK), :],
            out_sems[c % _NBUF],
        )

    in_cp = {}
    out_cp = {}
    for b in range(_NBUF - 1):
        in_cp[b] = start_in(b)
    for c in range(_NCHUNK):
        in_cp[c].wait()
        out_cp[c] = start_out(c)
        n = c + _NBUF - 1
        if n < _NCHUNK:
            if c >= 1:
                out_cp[c - 1].wait()
            in_cp[n] = start_in(n)
    for c in range(_NCHUNK - _NBUF, _NCHUNK):
        if c >= 0:
            out_cp[c].wait()


def kernel(x, embed_weight):
    return _copy_rows(embed_weight)
